# baseline (speedup 1.0000x reference)
import functools

import jax
import jax.numpy as jnp
from jax.experimental import pallas as pl
from jax.experimental.pallas import tpu as pltpu

_MAXD = 300
_LANES = 304
_R, _C = 160, 128
_NCHUNK = 8
_CROWS = _R // _NCHUNK
_CONF = 0.5
_IOU = 0.4


def _nms_body(x_ref, o_ref, s_ref, cls_ref, iou_ref, *, wf, hf):
    f32 = jnp.float32
    s = x_ref[4]
    cls = jnp.zeros((_R, _C), f32)
    for c in range(1, 80):
        xc = x_ref[4 + c]
        upd = xc > s
        s = jnp.where(upd, xc, s)
        cls = jnp.where(upd, f32(c), cls)
    cls_ref[...] = cls
    s_ref[...] = s

    lane = jax.lax.broadcasted_iota(jnp.int32, (1, _LANES), 1)
    lane128 = jax.lax.broadcasted_iota(jnp.int32, (1, _C), 1)
    flat20 = (jax.lax.broadcasted_iota(jnp.int32, (_CROWS, _C), 0) * _C
              + jax.lax.broadcasted_iota(jnp.int32, (_CROWS, _C), 1))
    zrow = jnp.zeros((1, _LANES), f32)

    cm = jnp.full((1, _C), -jnp.inf, f32)
    for j in range(_NCHUNK):
        cm = jnp.where(lane128 == j, jnp.max(s[j * _CROWS:(j + 1) * _CROWS, :]), cm)

    def sel_body(k, carry):
        cm, acc = carry
        m = jnp.max(cm)
        cid = jnp.min(jnp.where(cm == m, lane128, jnp.int32(1 << 20)))
        slab = s_ref[pl.ds(cid * _CROWS, _CROWS), :]
        fi = jnp.min(jnp.where(slab == m, flat20, jnp.int32(1 << 30)))
        rr = cid * _CROWS + fi // _C
        c = fi % _C
        cmask = lane128 == c
        rows5 = jnp.concatenate(
            [
                x_ref[0, pl.ds(rr, 1), :],
                x_ref[1, pl.ds(rr, 1), :],
                x_ref[2, pl.ds(rr, 1), :],
                x_ref[3, pl.ds(rr, 1), :],
                cls_ref[pl.ds(rr, 1), :],
            ],
            axis=0,
        )
        ext5 = jnp.sum(jnp.where(cmask, rows5, 0.0), axis=1, keepdims=True)
        col8 = jnp.concatenate(
            [ext5, jnp.full((1, 1), m, f32), jnp.zeros((2, 1), f32)], axis=0)
        acc = acc + col8 * (lane == k).astype(f32)
        newslab = jnp.where(flat20 == fi, -jnp.inf, slab)
        s_ref[pl.ds(cid * _CROWS, _CROWS), :] = newslab
        cm = jnp.where(lane128 == cid, jnp.max(newslab), cm)
        return cm, acc

    init = (cm, jnp.zeros((8, _LANES), f32))
    _, acc = jax.lax.fori_loop(0, _MAXD, sel_body, init)
    xr, yr, wr, hr = acc[0:1, :], acc[1:2, :], acc[2:3, :], acc[3:4, :]
    clsr, scr = acc[4:5, :], acc[5:6, :]

    x0r = xr - wr / 2
    y0r = yr - hr / 2
    x1r = xr + wr / 2
    y1r = yr + hr / 2
    ar = (x1r - x0r) * (y1r - y0r)
    valid = ((scr > _CONF) & (lane < _MAXD)).astype(f32)

    brows = jnp.concatenate([x0r, y0r, x1r, y1r, ar, zrow, zrow, zrow], axis=0)
    ident = (jax.lax.broadcasted_iota(jnp.int32, (_LANES, _LANES), 0)
             == jax.lax.broadcasted_iota(jnp.int32, (_LANES, _LANES), 1)
             ).astype(f32)
    cols = jax.lax.dot_general(
        ident, brows, (((1,), (1,)), ((), ())),
        preferred_element_type=f32, precision=jax.lax.Precision.HIGHEST)
    x0c, y0c = cols[:, 0:1], cols[:, 1:2]
    x1c, y1c, ac = cols[:, 2:3], cols[:, 3:4], cols[:, 4:5]

    iw = jnp.maximum(jnp.minimum(x1c, x1r) - jnp.maximum(x0c, x0r), 0.0)
    ih = jnp.maximum(jnp.minimum(y1c, y1r) - jnp.maximum(y0c, y0r), 0.0)
    inter = iw * ih
    iou_ref[...] = inter / (ac + ar - inter + 1e-9)

    def nms_body(i, kr):
        row = iou_ref[pl.ds(i, 1), :]
        ki = jnp.sum(kr * (lane == i).astype(f32))
        sup = ((row > _IOU) & (lane > i)).astype(f32) * ki
        return kr * (1.0 - sup)

    kr = jax.lax.fori_loop(0, _MAXD, nms_body, valid)

    rows = jnp.concatenate(
        [
            jnp.clip(x0r, 0.0, wf) * kr,
            jnp.clip(y0r, 0.0, hf) * kr,
            jnp.clip(x1r, 0.0, wf) * kr,
            jnp.clip(y1r, 0.0, hf) * kr,
            scr * kr,
            clsr * kr,
            zrow,
            zrow,
        ],
        axis=0,
    )
    o_ref[...] = rows


def kernel(pred, orig_img):
    H = orig_img.shape[1]
    W = orig_img.shape[2]
    p = pred[0]
    n = p.shape[1]
    pad = _R * _C - n
    xp = jnp.concatenate(
        [
            jnp.pad(p[:4], ((0, 0), (0, pad))),
            jnp.pad(p[4:], ((0, 0), (0, pad)), constant_values=-jnp.inf),
        ],
        axis=0,
    ).reshape(84, _R, _C)
    out = pl.pallas_call(
        functools.partial(_nms_body, wf=float(W), hf=float(H)),
        out_shape=jax.ShapeDtypeStruct((8, _LANES), jnp.float32),
        scratch_shapes=[
            pltpu.VMEM((_R, _C), jnp.float32),
            pltpu.VMEM((_R, _C), jnp.float32),
            pltpu.VMEM((_LANES, _LANES), jnp.float32),
        ],
    )(xp)
    return out[:6, :_MAXD].T


# speedup vs baseline: 1.2207x; 1.2207x over previous
import functools

import jax
import jax.numpy as jnp
from jax.experimental import pallas as pl
from jax.experimental.pallas import tpu as pltpu

_MAXD = 300
_LANES = 304
_R, _C = 160, 128
_NCHUNK = 8
_CROWS = _R // _NCHUNK
_CONF = 0.5
_IOU = 0.4


def _nms_body(x_ref, o_ref, cls_ref, iou_ref, *, wf, hf):
    f32 = jnp.float32
    s = x_ref[4]
    cls = jnp.zeros((_R, _C), f32)
    for c in range(1, 80):
        xc = x_ref[4 + c]
        upd = xc > s
        s = jnp.where(upd, xc, s)
        cls = jnp.where(upd, f32(c), cls)
    cls_ref[...] = cls

    lane = jax.lax.broadcasted_iota(jnp.int32, (1, _LANES), 1)
    lane128 = jax.lax.broadcasted_iota(jnp.int32, (1, _C), 1)
    flat = (jax.lax.broadcasted_iota(jnp.int32, (_R, _C), 0) * _C
            + jax.lax.broadcasted_iota(jnp.int32, (_R, _C), 1))
    zrow = jnp.zeros((1, _LANES), f32)

    def sel_body(k, carry):
        s, acc = carry
        m = jnp.max(s)
        fi = jnp.min(jnp.where(s == m, flat, jnp.int32(1 << 30)))
        rr = fi // _C
        c = fi % _C
        cmask = lane128 == c
        rows5 = jnp.concatenate(
            [
                x_ref[0, pl.ds(rr, 1), :],
                x_ref[1, pl.ds(rr, 1), :],
                x_ref[2, pl.ds(rr, 1), :],
                x_ref[3, pl.ds(rr, 1), :],
                cls_ref[pl.ds(rr, 1), :],
            ],
            axis=0,
        )
        ext5 = jnp.sum(jnp.where(cmask, rows5, 0.0), axis=1, keepdims=True)
        col8 = jnp.concatenate(
            [ext5, jnp.full((1, 1), m, f32), jnp.zeros((2, 1), f32)], axis=0)
        acc = acc + col8 * (lane == k).astype(f32)
        s = jnp.where(flat == fi, -jnp.inf, s)
        return s, acc

    init = (s, jnp.zeros((8, _LANES), f32))
    _, acc = jax.lax.fori_loop(0, _MAXD, sel_body, init)
    xr, yr, wr, hr = acc[0:1, :], acc[1:2, :], acc[2:3, :], acc[3:4, :]
    clsr, scr = acc[4:5, :], acc[5:6, :]

    x0r = xr - wr / 2
    y0r = yr - hr / 2
    x1r = xr + wr / 2
    y1r = yr + hr / 2
    ar = (x1r - x0r) * (y1r - y0r)
    valid = ((scr > _CONF) & (lane < _MAXD)).astype(f32)

    brows = jnp.concatenate([x0r, y0r, x1r, y1r, ar, zrow, zrow, zrow], axis=0)
    ident = (jax.lax.broadcasted_iota(jnp.int32, (_LANES, _LANES), 0)
             == jax.lax.broadcasted_iota(jnp.int32, (_LANES, _LANES), 1)
             ).astype(f32)
    cols = jax.lax.dot_general(
        ident, brows, (((1,), (1,)), ((), ())),
        preferred_element_type=f32, precision=jax.lax.Precision.HIGHEST)
    x0c, y0c = cols[:, 0:1], cols[:, 1:2]
    x1c, y1c, ac = cols[:, 2:3], cols[:, 3:4], cols[:, 4:5]

    iw = jnp.maximum(jnp.minimum(x1c, x1r) - jnp.maximum(x0c, x0r), 0.0)
    ih = jnp.maximum(jnp.minimum(y1c, y1r) - jnp.maximum(y0c, y0r), 0.0)
    inter = iw * ih
    iou_ref[...] = inter / (ac + ar - inter + 1e-9)

    def nms_body(i, kr):
        row = iou_ref[pl.ds(i, 1), :]
        ki = jnp.sum(kr * (lane == i).astype(f32))
        sup = ((row > _IOU) & (lane > i)).astype(f32) * ki
        return kr * (1.0 - sup)

    kr = jax.lax.fori_loop(0, _MAXD, nms_body, valid)

    rows = jnp.concatenate(
        [
            jnp.clip(x0r, 0.0, wf) * kr,
            jnp.clip(y0r, 0.0, hf) * kr,
            jnp.clip(x1r, 0.0, wf) * kr,
            jnp.clip(y1r, 0.0, hf) * kr,
            scr * kr,
            clsr * kr,
            zrow,
            zrow,
        ],
        axis=0,
    )
    o_ref[...] = rows


def kernel(pred, orig_img):
    H = orig_img.shape[1]
    W = orig_img.shape[2]
    p = pred[0]
    n = p.shape[1]
    pad = _R * _C - n
    xp = jnp.concatenate(
        [
            jnp.pad(p[:4], ((0, 0), (0, pad))),
            jnp.pad(p[4:], ((0, 0), (0, pad)), constant_values=-jnp.inf),
        ],
        axis=0,
    ).reshape(84, _R, _C)
    out = pl.pallas_call(
        functools.partial(_nms_body, wf=float(W), hf=float(H)),
        out_shape=jax.ShapeDtypeStruct((8, _LANES), jnp.float32),
        scratch_shapes=[
            pltpu.VMEM((_R, _C), jnp.float32),
            pltpu.VMEM((_LANES, _LANES), jnp.float32),
        ],
    )(xp)
    return out[:6, :_MAXD].T


# speedup vs baseline: 1.4098x; 1.1549x over previous
import functools

import jax
import jax.numpy as jnp
from jax.experimental import pallas as pl
from jax.experimental.pallas import tpu as pltpu

_MAXD = 300
_LANES = 304
_R, _C = 160, 128
_NCHUNK = 8
_CROWS = _R // _NCHUNK
_CONF = 0.5
_IOU = 0.4


def _nms_body(x_ref, o_ref, cls_ref, iou_ref, *, wf, hf):
    f32 = jnp.float32
    s = x_ref[4]
    cls = jnp.zeros((_R, _C), f32)
    for c in range(1, 80):
        xc = x_ref[4 + c]
        upd = xc > s
        s = jnp.where(upd, xc, s)
        cls = jnp.where(upd, f32(c), cls)
    cls_ref[...] = cls

    lane = jax.lax.broadcasted_iota(jnp.int32, (1, _LANES), 1)
    lane128 = jax.lax.broadcasted_iota(jnp.int32, (1, _C), 1)
    flat = (jax.lax.broadcasted_iota(jnp.int32, (_R, _C), 0) * _C
            + jax.lax.broadcasted_iota(jnp.int32, (_R, _C), 1))
    zrow = jnp.zeros((1, _LANES), f32)

    def sel_step(k, s, acc):
        m = jnp.max(s)
        fi = jnp.min(jnp.where(s == m, flat, jnp.int32(1 << 30)))
        rr = fi // _C
        c = fi % _C
        cmask = lane128 == c
        rows5 = jnp.concatenate(
            [
                x_ref[0, pl.ds(rr, 1), :],
                x_ref[1, pl.ds(rr, 1), :],
                x_ref[2, pl.ds(rr, 1), :],
                x_ref[3, pl.ds(rr, 1), :],
                cls_ref[pl.ds(rr, 1), :],
            ],
            axis=0,
        )
        ext5 = jnp.sum(jnp.where(cmask, rows5, 0.0), axis=1, keepdims=True)
        col8 = jnp.concatenate(
            [ext5, jnp.full((1, 1), m, f32), jnp.zeros((2, 1), f32)], axis=0)
        acc = acc + col8 * (lane == k).astype(f32)
        s = jnp.where(flat == fi, -jnp.inf, s)
        return s, acc

    def sel_body(k4, carry):
        s, acc = carry
        for u in range(4):
            s, acc = sel_step(k4 * 4 + u, s, acc)
        return s, acc

    init = (s, jnp.zeros((8, _LANES), f32))
    _, acc = jax.lax.fori_loop(0, _MAXD // 4, sel_body, init)
    xr, yr, wr, hr = acc[0:1, :], acc[1:2, :], acc[2:3, :], acc[3:4, :]
    clsr, scr = acc[4:5, :], acc[5:6, :]

    x0r = xr - wr / 2
    y0r = yr - hr / 2
    x1r = xr + wr / 2
    y1r = yr + hr / 2
    ar = (x1r - x0r) * (y1r - y0r)
    valid = ((scr > _CONF) & (lane < _MAXD)).astype(f32)

    brows = jnp.concatenate([x0r, y0r, x1r, y1r, ar, zrow, zrow, zrow], axis=0)
    ident = (jax.lax.broadcasted_iota(jnp.int32, (_LANES, _LANES), 0)
             == jax.lax.broadcasted_iota(jnp.int32, (_LANES, _LANES), 1)
             ).astype(f32)
    cols = jax.lax.dot_general(
        ident, brows, (((1,), (1,)), ((), ())),
        preferred_element_type=f32, precision=jax.lax.Precision.HIGHEST)
    x0c, y0c = cols[:, 0:1], cols[:, 1:2]
    x1c, y1c, ac = cols[:, 2:3], cols[:, 3:4], cols[:, 4:5]

    iw = jnp.maximum(jnp.minimum(x1c, x1r) - jnp.maximum(x0c, x0r), 0.0)
    ih = jnp.maximum(jnp.minimum(y1c, y1r) - jnp.maximum(y0c, y0r), 0.0)
    inter = iw * ih
    iou_ref[...] = inter / (ac + ar - inter + 1e-9)

    def nms_step(i, kr):
        row = iou_ref[pl.ds(i, 1), :]
        ki = jnp.sum(kr * (lane == i).astype(f32))
        sup = ((row > _IOU) & (lane > i)).astype(f32) * ki
        return kr * (1.0 - sup)

    def nms_body(i4, kr):
        for u in range(4):
            kr = nms_step(i4 * 4 + u, kr)
        return kr

    kr = jax.lax.fori_loop(0, _MAXD // 4, nms_body, valid)

    rows = jnp.concatenate(
        [
            jnp.clip(x0r, 0.0, wf) * kr,
            jnp.clip(y0r, 0.0, hf) * kr,
            jnp.clip(x1r, 0.0, wf) * kr,
            jnp.clip(y1r, 0.0, hf) * kr,
            scr * kr,
            clsr * kr,
            zrow,
            zrow,
        ],
        axis=0,
    )
    o_ref[...] = rows


def kernel(pred, orig_img):
    H = orig_img.shape[1]
    W = orig_img.shape[2]
    p = pred[0]
    n = p.shape[1]
    pad = _R * _C - n
    xp = jnp.concatenate(
        [
            jnp.pad(p[:4], ((0, 0), (0, pad))),
            jnp.pad(p[4:], ((0, 0), (0, pad)), constant_values=-jnp.inf),
        ],
        axis=0,
    ).reshape(84, _R, _C)
    out = pl.pallas_call(
        functools.partial(_nms_body, wf=float(W), hf=float(H)),
        out_shape=jax.ShapeDtypeStruct((8, _LANES), jnp.float32),
        scratch_shapes=[
            pltpu.VMEM((_R, _C), jnp.float32),
            pltpu.VMEM((_LANES, _LANES), jnp.float32),
        ],
    )(xp)
    return out[:6, :_MAXD].T


# speedup vs baseline: 1.4438x; 1.0241x over previous
import functools

import jax
import jax.numpy as jnp
from jax import lax
from jax.experimental import pallas as pl
from jax.experimental.pallas import tpu as pltpu
from jax.experimental.pallas import tpu_sc as plsc

_MAXD = 300
_LANES = 304
_R, _C = 160, 128
_CONF = 0.5
_IOU = 0.4
_NC, _NS = 2, 16


def _select_body(x_ref, sel_ref, idx_ref, cls_ref):
    f32 = jnp.float32
    s = x_ref[4]
    cls = jnp.zeros((_R, _C), f32)
    for c in range(1, 80):
        xc = x_ref[4 + c]
        upd = xc > s
        s = jnp.where(upd, xc, s)
        cls = jnp.where(upd, f32(c), cls)
    cls_ref[...] = cls

    lane = jax.lax.broadcasted_iota(jnp.int32, (1, _LANES), 1)
    lane128 = jax.lax.broadcasted_iota(jnp.int32, (3, _C), 1)
    sub3 = jax.lax.broadcasted_iota(jnp.int32, (3, _C), 0)
    flat = (jax.lax.broadcasted_iota(jnp.int32, (_R, _C), 0) * _C
            + jax.lax.broadcasted_iota(jnp.int32, (_R, _C), 1))

    def sel_step(k, s, scr, idxa):
        m = jnp.max(s)
        fi = jnp.min(jnp.where(s == m, flat, jnp.int32(1 << 30)))
        scr = scr + m * (lane == k).astype(f32)
        mask3 = (sub3 == k // _C) & (lane128 == k % _C)
        idxa = idxa + fi * mask3.astype(jnp.int32)
        s = jnp.where(flat == fi, -jnp.inf, s)
        return s, scr, idxa

    def sel_body(k4, carry):
        s, scr, idxa = carry
        for u in range(4):
            s, scr, idxa = sel_step(k4 * 4 + u, s, scr, idxa)
        return s, scr, idxa

    init = (s, jnp.zeros((1, _LANES), f32), jnp.zeros((3, _C), jnp.int32))
    _, scr, idxa = jax.lax.fori_loop(0, _MAXD // 4, sel_body, init)
    sel_ref[...] = scr
    idx_ref[...] = idxa


def _sc_gather_body(x0h, x1h, x2h, x3h, clsh, idxh, out, idx_v, g_v, sem):
    wid = lax.axis_index("s") * _NC + lax.axis_index("c")
    for t, src in enumerate((x0h, x1h, x2h, x3h, clsh)):
        @pl.when(wid == t)
        def _():
            pltpu.sync_copy(idxh, idx_v)
            for j in range(3):
                pltpu.async_copy(src.at[idx_v.at[j]], g_v.at[j], sem).wait()
            pltpu.sync_copy(g_v, out.at[t])


def _nms_body(sel_ref, g_ref, o_ref, iou_ref, *, wf, hf):
    f32 = jnp.float32
    lane = jax.lax.broadcasted_iota(jnp.int32, (1, _LANES), 1)
    zrow = jnp.zeros((1, _LANES), f32)
    scr = sel_ref[...]
    xr = g_ref[0:1, :_LANES]
    yr = g_ref[1:2, :_LANES]
    wr = g_ref[2:3, :_LANES]
    hr = g_ref[3:4, :_LANES]
    clsr = g_ref[4:5, :_LANES]

    x0r = xr - wr / 2
    y0r = yr - hr / 2
    x1r = xr + wr / 2
    y1r = yr + hr / 2
    ar = (x1r - x0r) * (y1r - y0r)
    valid = ((scr > _CONF) & (lane < _MAXD)).astype(f32)

    brows = jnp.concatenate([x0r, y0r, x1r, y1r, ar, zrow, zrow, zrow], axis=0)
    ident = (jax.lax.broadcasted_iota(jnp.int32, (_LANES, _LANES), 0)
             == jax.lax.broadcasted_iota(jnp.int32, (_LANES, _LANES), 1)
             ).astype(f32)
    cols = jax.lax.dot_general(
        ident, brows, (((1,), (1,)), ((), ())),
        preferred_element_type=f32, precision=jax.lax.Precision.HIGHEST)
    x0c, y0c = cols[:, 0:1], cols[:, 1:2]
    x1c, y1c, ac = cols[:, 2:3], cols[:, 3:4], cols[:, 4:5]

    iw = jnp.maximum(jnp.minimum(x1c, x1r) - jnp.maximum(x0c, x0r), 0.0)
    ih = jnp.maximum(jnp.minimum(y1c, y1r) - jnp.maximum(y0c, y0r), 0.0)
    inter = iw * ih
    iou_ref[...] = inter / (ac + ar - inter + 1e-9)

    def nms_step(i, kr):
        row = iou_ref[pl.ds(i, 1), :]
        ki = jnp.sum(kr * (lane == i).astype(f32))
        sup = ((row > _IOU) & (lane > i)).astype(f32) * ki
        return kr * (1.0 - sup)

    def nms_body(i4, kr):
        for u in range(4):
            kr = nms_step(i4 * 4 + u, kr)
        return kr

    kr = jax.lax.fori_loop(0, _MAXD // 4, nms_body, valid)

    rows = jnp.concatenate(
        [
            jnp.clip(x0r, 0.0, wf) * kr,
            jnp.clip(y0r, 0.0, hf) * kr,
            jnp.clip(x1r, 0.0, wf) * kr,
            jnp.clip(y1r, 0.0, hf) * kr,
            scr * kr,
            clsr * kr,
            zrow,
            zrow,
        ],
        axis=0,
    )
    o_ref[...] = rows


def kernel(pred, orig_img):
    H = orig_img.shape[1]
    W = orig_img.shape[2]
    p = pred[0]
    n = p.shape[1]
    pad = _R * _C - n
    xb = jnp.concatenate(
        [
            jnp.pad(p[:4], ((0, 0), (0, pad))),
            jnp.pad(p[4:], ((0, 0), (0, pad)), constant_values=-jnp.inf),
        ],
        axis=0,
    )
    xp = xb.reshape(84, _R, _C)

    sel, idx3, clsp = pl.pallas_call(
        _select_body,
        out_shape=[
            jax.ShapeDtypeStruct((1, _LANES), jnp.float32),
            jax.ShapeDtypeStruct((3, _C), jnp.int32),
            jax.ShapeDtypeStruct((_R, _C), jnp.float32),
        ],
    )(xp)

    mesh = plsc.VectorSubcoreMesh(core_axis_name="c", subcore_axis_name="s")
    gath = functools.partial(
        pl.kernel,
        mesh=mesh,
        out_type=jax.ShapeDtypeStruct((5, 3, _C), jnp.float32),
        scratch_types=[
            pltpu.VMEM((3, _C), jnp.int32),
            pltpu.VMEM((3, _C), jnp.float32),
            pltpu.SemaphoreType.DMA,
        ],
    )(_sc_gather_body)(
        xb[0], xb[1], xb[2], xb[3], clsp.reshape(_R * _C), idx3)

    out = pl.pallas_call(
        functools.partial(_nms_body, wf=float(W), hf=float(H)),
        out_shape=jax.ShapeDtypeStruct((8, _LANES), jnp.float32),
        scratch_shapes=[pltpu.VMEM((_LANES, _LANES), jnp.float32)],
    )(sel, gath.reshape(5, 3 * _C))
    return out[:6, :_MAXD].T


# speedup vs baseline: 1.7191x; 1.1907x over previous
import functools

import jax
import jax.numpy as jnp
from jax import lax
from jax.experimental import pallas as pl
from jax.experimental.pallas import tpu as pltpu
from jax.experimental.pallas import tpu_sc as plsc

_MAXD = 300
_LANES = 304
_R, _C = 160, 128
_CONF = 0.5
_IOU = 0.4
_NC, _NS = 2, 16


def _select_body(x_ref, sel_ref, idx_ref, cls_ref):
    f32 = jnp.float32
    s = x_ref[4]
    cls = jnp.zeros((_R, _C), f32)
    for c in range(1, 80):
        xc = x_ref[4 + c]
        upd = xc > s
        s = jnp.where(upd, xc, s)
        cls = jnp.where(upd, f32(c), cls)
    cls_ref[...] = cls

    lane = jax.lax.broadcasted_iota(jnp.int32, (1, _LANES), 1)
    lane128 = jax.lax.broadcasted_iota(jnp.int32, (3, _C), 1)
    sub3 = jax.lax.broadcasted_iota(jnp.int32, (3, _C), 0)
    lane128r = jax.lax.broadcasted_iota(jnp.int32, (1, _C), 1)
    row160 = jax.lax.broadcasted_iota(jnp.int32, (_R, _C), 0)
    lane2d = jax.lax.broadcasted_iota(jnp.int32, (_R, _C), 1)
    flat = row160 * _C + lane2d
    big = jnp.int32(1 << 30)

    cm = jnp.max(s, axis=0, keepdims=True)
    rw = jnp.min(jnp.where(s == cm, row160, big), axis=0, keepdims=True)

    def sel_step(k, s, cm, rw, scr, idxa):
        m = jnp.max(cm, axis=1, keepdims=True)
        fi = jnp.min(jnp.where(cm == m, rw * _C + lane128r, big),
                     axis=1, keepdims=True)
        scr = scr + jnp.where(lane == k, m, 0.0)
        mask3 = (sub3 == k // _C) & (lane128 == k % _C)
        idxa = idxa + jnp.where(mask3, fi, 0)
        s = jnp.where(flat == fi, -jnp.inf, s)
        cvec = lane2d == fi % _C
        colv = jnp.where(cvec, s, -jnp.inf)
        ncm = jnp.max(colv, axis=0, keepdims=True)
        nrw = jnp.min(jnp.where(colv == ncm, row160, big), axis=0, keepdims=True)
        cl = lane128r == fi % _C
        cm = jnp.where(cl, ncm, cm)
        rw = jnp.where(cl, nrw, rw)
        return s, cm, rw, scr, idxa

    def sel_body(k4, carry):
        s, cm, rw, scr, idxa = carry
        for u in range(4):
            s, cm, rw, scr, idxa = sel_step(k4 * 4 + u, s, cm, rw, scr, idxa)
        return s, cm, rw, scr, idxa

    init = (s, cm, rw, jnp.zeros((1, _LANES), f32), jnp.zeros((3, _C), jnp.int32))
    _, _, _, scr, idxa = jax.lax.fori_loop(0, _MAXD // 4, sel_body, init)
    sel_ref[...] = scr
    idx_ref[...] = idxa


def _sc_gather_body(x0h, x1h, x2h, x3h, clsh, idxh, out, idx_v, g_v, sem):
    wid = lax.axis_index("s") * _NC + lax.axis_index("c")
    for t, src in enumerate((x0h, x1h, x2h, x3h, clsh)):
        @pl.when(wid == t)
        def _():
            pltpu.sync_copy(idxh, idx_v)
            for j in range(3):
                pltpu.async_copy(src.at[idx_v.at[j]], g_v.at[j], sem).wait()
            pltpu.sync_copy(g_v, out.at[t])


def _nms_body(sel_ref, g_ref, o_ref, iou_ref, *, wf, hf):
    f32 = jnp.float32
    lane = jax.lax.broadcasted_iota(jnp.int32, (1, _LANES), 1)
    zrow = jnp.zeros((1, _LANES), f32)
    scr = sel_ref[...]
    xr = g_ref[0:1, :_LANES]
    yr = g_ref[1:2, :_LANES]
    wr = g_ref[2:3, :_LANES]
    hr = g_ref[3:4, :_LANES]
    clsr = g_ref[4:5, :_LANES]

    x0r = xr - wr / 2
    y0r = yr - hr / 2
    x1r = xr + wr / 2
    y1r = yr + hr / 2
    ar = (x1r - x0r) * (y1r - y0r)
    valid = ((scr > _CONF) & (lane < _MAXD)).astype(f32)

    brows = jnp.concatenate([x0r, y0r, x1r, y1r, ar, zrow, zrow, zrow], axis=0)
    ident = (jax.lax.broadcasted_iota(jnp.int32, (_LANES, _LANES), 0)
             == jax.lax.broadcasted_iota(jnp.int32, (_LANES, _LANES), 1)
             ).astype(f32)
    cols = jax.lax.dot_general(
        ident, brows, (((1,), (1,)), ((), ())),
        preferred_element_type=f32, precision=jax.lax.Precision.HIGHEST)
    x0c, y0c = cols[:, 0:1], cols[:, 1:2]
    x1c, y1c, ac = cols[:, 2:3], cols[:, 3:4], cols[:, 4:5]

    iw = jnp.maximum(jnp.minimum(x1c, x1r) - jnp.maximum(x0c, x0r), 0.0)
    ih = jnp.maximum(jnp.minimum(y1c, y1r) - jnp.maximum(y0c, y0r), 0.0)
    inter = iw * ih
    iou_ref[...] = inter / (ac + ar - inter + 1e-9)

    def nms_step(i, kr):
        row = iou_ref[pl.ds(i, 1), :]
        ki = jnp.sum(kr * (lane == i).astype(f32), axis=1, keepdims=True)
        sup = ((row > _IOU) & (lane > i)).astype(f32) * ki
        return kr * (1.0 - sup)

    def nms_body(i4, kr):
        for u in range(4):
            kr = nms_step(i4 * 4 + u, kr)
        return kr

    kr = jax.lax.fori_loop(0, _MAXD // 4, nms_body, valid)

    rows = jnp.concatenate(
        [
            jnp.clip(x0r, 0.0, wf) * kr,
            jnp.clip(y0r, 0.0, hf) * kr,
            jnp.clip(x1r, 0.0, wf) * kr,
            jnp.clip(y1r, 0.0, hf) * kr,
            scr * kr,
            clsr * kr,
            zrow,
            zrow,
        ],
        axis=0,
    )
    o_ref[...] = rows


def kernel(pred, orig_img):
    H = orig_img.shape[1]
    W = orig_img.shape[2]
    p = pred[0]
    n = p.shape[1]
    pad = _R * _C - n
    xb = jnp.concatenate(
        [
            jnp.pad(p[:4], ((0, 0), (0, pad))),
            jnp.pad(p[4:], ((0, 0), (0, pad)), constant_values=-jnp.inf),
        ],
        axis=0,
    )
    xp = xb.reshape(84, _R, _C)

    sel, idx3, clsp = pl.pallas_call(
        _select_body,
        out_shape=[
            jax.ShapeDtypeStruct((1, _LANES), jnp.float32),
            jax.ShapeDtypeStruct((3, _C), jnp.int32),
            jax.ShapeDtypeStruct((_R, _C), jnp.float32),
        ],
    )(xp)

    mesh = plsc.VectorSubcoreMesh(core_axis_name="c", subcore_axis_name="s")
    gath = functools.partial(
        pl.kernel,
        mesh=mesh,
        out_type=jax.ShapeDtypeStruct((5, 3, _C), jnp.float32),
        scratch_types=[
            pltpu.VMEM((3, _C), jnp.int32),
            pltpu.VMEM((3, _C), jnp.float32),
            pltpu.SemaphoreType.DMA,
        ],
    )(_sc_gather_body)(
        xb[0], xb[1], xb[2], xb[3], clsp.reshape(_R * _C), idx3)

    out = pl.pallas_call(
        functools.partial(_nms_body, wf=float(W), hf=float(H)),
        out_shape=jax.ShapeDtypeStruct((8, _LANES), jnp.float32),
        scratch_shapes=[pltpu.VMEM((_LANES, _LANES), jnp.float32)],
    )(sel, gath.reshape(5, 3 * _C))
    return out[:6, :_MAXD].T


# speedup vs baseline: 1.7278x; 1.0050x over previous
import functools

import jax
import jax.numpy as jnp
from jax import lax
from jax.experimental import pallas as pl
from jax.experimental.pallas import tpu as pltpu
from jax.experimental.pallas import tpu_sc as plsc

_MAXD = 300
_LANES = 304
_R, _C = 160, 128
_CONF = 0.5
_IOU = 0.4
_NC, _NS = 2, 16


def _select_body(x_ref, sel_ref, idx_ref, cls_ref):
    f32 = jnp.float32
    s = x_ref[4]
    cls = jnp.zeros((_R, _C), f32)
    for c in range(1, 80):
        xc = x_ref[4 + c]
        upd = xc > s
        s = jnp.where(upd, xc, s)
        cls = jnp.where(upd, f32(c), cls)
    cls_ref[...] = cls

    lane = jax.lax.broadcasted_iota(jnp.int32, (1, _LANES), 1)
    lane128 = jax.lax.broadcasted_iota(jnp.int32, (3, _C), 1)
    sub3 = jax.lax.broadcasted_iota(jnp.int32, (3, _C), 0)
    lane128r = jax.lax.broadcasted_iota(jnp.int32, (1, _C), 1)
    row160 = jax.lax.broadcasted_iota(jnp.int32, (_R, _C), 0)
    lane2d = jax.lax.broadcasted_iota(jnp.int32, (_R, _C), 1)
    flat = row160 * _C + lane2d
    big = jnp.int32(1 << 30)

    cm = jnp.max(s, axis=0, keepdims=True)
    rw = jnp.min(jnp.where(s == cm, row160, big), axis=0, keepdims=True)

    def sel_step(k, s, cm, rw, scr, idxa):
        m = jnp.max(cm, axis=1, keepdims=True)
        fi = jnp.min(jnp.where(cm == m, rw * _C + lane128r, big),
                     axis=1, keepdims=True)
        scr = scr + jnp.where(lane == k, m, 0.0)
        mask3 = (sub3 == k // _C) & (lane128 == k % _C)
        idxa = idxa + jnp.where(mask3, fi, 0)
        s = jnp.where(flat == fi, -jnp.inf, s)
        cvec = lane2d == fi % _C
        colv = jnp.where(cvec, s, -jnp.inf)
        ncm = jnp.max(colv, axis=0, keepdims=True)
        nrw = jnp.min(jnp.where(colv == ncm, row160, big), axis=0, keepdims=True)
        cl = lane128r == fi % _C
        cm = jnp.where(cl, ncm, cm)
        rw = jnp.where(cl, nrw, rw)
        return s, cm, rw, scr, idxa

    def sel_body(k4, carry):
        s, cm, rw, scr, idxa = carry
        for u in range(10):
            s, cm, rw, scr, idxa = sel_step(k4 * 10 + u, s, cm, rw, scr, idxa)
        return s, cm, rw, scr, idxa

    init = (s, cm, rw, jnp.zeros((1, _LANES), f32), jnp.zeros((3, _C), jnp.int32))
    _, _, _, scr, idxa = jax.lax.fori_loop(0, _MAXD // 10, sel_body, init)
    sel_ref[...] = scr
    idx_ref[...] = idxa


def _sc_gather_body(x0h, x1h, x2h, x3h, clsh, idxh, out, idx_v, g_v, sem):
    wid = lax.axis_index("s") * _NC + lax.axis_index("c")
    for t, src in enumerate((x0h, x1h, x2h, x3h, clsh)):
        @pl.when(wid == t)
        def _():
            pltpu.sync_copy(idxh, idx_v)
            for j in range(3):
                pltpu.async_copy(src.at[idx_v.at[j]], g_v.at[j], sem).wait()
            pltpu.sync_copy(g_v, out.at[t])


def _nms_body(sel_ref, g_ref, o_ref, iou_ref, *, wf, hf):
    f32 = jnp.float32
    lane = jax.lax.broadcasted_iota(jnp.int32, (1, _LANES), 1)
    zrow = jnp.zeros((1, _LANES), f32)
    scr = sel_ref[...]
    xr = g_ref[0:1, :_LANES]
    yr = g_ref[1:2, :_LANES]
    wr = g_ref[2:3, :_LANES]
    hr = g_ref[3:4, :_LANES]
    clsr = g_ref[4:5, :_LANES]

    x0r = xr - wr / 2
    y0r = yr - hr / 2
    x1r = xr + wr / 2
    y1r = yr + hr / 2
    ar = (x1r - x0r) * (y1r - y0r)
    valid = ((scr > _CONF) & (lane < _MAXD)).astype(f32)

    brows = jnp.concatenate([x0r, y0r, x1r, y1r, ar, zrow, zrow, zrow], axis=0)
    ident = (jax.lax.broadcasted_iota(jnp.int32, (_LANES, _LANES), 0)
             == jax.lax.broadcasted_iota(jnp.int32, (_LANES, _LANES), 1)
             ).astype(f32)
    cols = jax.lax.dot_general(
        ident, brows, (((1,), (1,)), ((), ())),
        preferred_element_type=f32, precision=jax.lax.Precision.HIGHEST)
    x0c, y0c = cols[:, 0:1], cols[:, 1:2]
    x1c, y1c, ac = cols[:, 2:3], cols[:, 3:4], cols[:, 4:5]

    iw = jnp.maximum(jnp.minimum(x1c, x1r) - jnp.maximum(x0c, x0r), 0.0)
    ih = jnp.maximum(jnp.minimum(y1c, y1r) - jnp.maximum(y0c, y0r), 0.0)
    inter = iw * ih
    iou_ref[...] = inter / (ac + ar - inter + 1e-9)

    def nms_step(i, kr):
        row = iou_ref[pl.ds(i, 1), :]
        ki = jnp.sum(kr * (lane == i).astype(f32), axis=1, keepdims=True)
        sup = ((row > _IOU) & (lane > i)).astype(f32) * ki
        return kr * (1.0 - sup)

    def nms_body(i4, kr):
        for u in range(10):
            kr = nms_step(i4 * 10 + u, kr)
        return kr

    kr = jax.lax.fori_loop(0, _MAXD // 10, nms_body, valid)

    rows = jnp.concatenate(
        [
            jnp.clip(x0r, 0.0, wf) * kr,
            jnp.clip(y0r, 0.0, hf) * kr,
            jnp.clip(x1r, 0.0, wf) * kr,
            jnp.clip(y1r, 0.0, hf) * kr,
            scr * kr,
            clsr * kr,
            zrow,
            zrow,
        ],
        axis=0,
    )
    o_ref[...] = rows


def kernel(pred, orig_img):
    H = orig_img.shape[1]
    W = orig_img.shape[2]
    p = pred[0]
    n = p.shape[1]
    pad = _R * _C - n
    xb = jnp.concatenate(
        [
            jnp.pad(p[:4], ((0, 0), (0, pad))),
            jnp.pad(p[4:], ((0, 0), (0, pad)), constant_values=-jnp.inf),
        ],
        axis=0,
    )
    xp = xb.reshape(84, _R, _C)

    sel, idx3, clsp = pl.pallas_call(
        _select_body,
        out_shape=[
            jax.ShapeDtypeStruct((1, _LANES), jnp.float32),
            jax.ShapeDtypeStruct((3, _C), jnp.int32),
            jax.ShapeDtypeStruct((_R, _C), jnp.float32),
        ],
    )(xp)

    mesh = plsc.VectorSubcoreMesh(core_axis_name="c", subcore_axis_name="s")
    gath = functools.partial(
        pl.kernel,
        mesh=mesh,
        out_type=jax.ShapeDtypeStruct((5, 3, _C), jnp.float32),
        scratch_types=[
            pltpu.VMEM((3, _C), jnp.int32),
            pltpu.VMEM((3, _C), jnp.float32),
            pltpu.SemaphoreType.DMA,
        ],
    )(_sc_gather_body)(
        xb[0], xb[1], xb[2], xb[3], clsp.reshape(_R * _C), idx3)

    out = pl.pallas_call(
        functools.partial(_nms_body, wf=float(W), hf=float(H)),
        out_shape=jax.ShapeDtypeStruct((8, _LANES), jnp.float32),
        scratch_shapes=[pltpu.VMEM((_LANES, _LANES), jnp.float32)],
    )(sel, gath.reshape(5, 3 * _C))
    return out[:6, :_MAXD].T
